# initial kernel scaffold (unmeasured)
import jax
import jax.numpy as jnp
from jax import lax
from jax.experimental import pallas as pl
from jax.experimental.pallas import tpu as pltpu


def _exchange_add(partial):
    t, d = partial.shape

    def body(p_ref, out_ref, comm_ref, send_sem, recv_sem):
        my_x = lax.axis_index("x")
        my_y = lax.axis_index("y")
        my_z = lax.axis_index("z")
        nbr = (my_x, 1 - my_y, my_z)

        barrier_sem = pltpu.get_barrier_semaphore()
        pl.semaphore_signal(
            barrier_sem, inc=1, device_id=nbr,
            device_id_type=pl.DeviceIdType.MESH,
        )
        pl.semaphore_wait(barrier_sem, 1)

        rdma = pltpu.make_async_remote_copy(
            src_ref=p_ref,
            dst_ref=comm_ref,
            send_sem=send_sem,
            recv_sem=recv_sem,
            device_id=nbr,
            device_id_type=pl.DeviceIdType.MESH,
        )
        rdma.start()
        rdma.wait()

        out_ref[...] = p_ref[...] + comm_ref[...]

    return pl.pallas_call(
        body,
        out_shape=jax.ShapeDtypeStruct((t, d), jnp.float32),
        in_specs=[pl.BlockSpec(memory_space=pltpu.VMEM)],
        out_specs=pl.BlockSpec(memory_space=pltpu.VMEM),
        scratch_shapes=[
            pltpu.VMEM((t, d), jnp.float32),
            pltpu.SemaphoreType.DMA,
            pltpu.SemaphoreType.DMA,
        ],
        compiler_params=pltpu.CompilerParams(collective_id=0),
    )(partial)


def kernel(ids, E):
    my_y = lax.axis_index("y")
    v_shard = E.shape[0]
    local = ids - my_y * v_shard
    in_range = (local >= 0) & (local < v_shard)
    safe = jnp.where(in_range, local, 0)
    partial = jnp.where(in_range[:, None], E[safe], 0.0).astype(jnp.float32)
    return _exchange_add(partial)


# baseline (device time: 156656 ns/iter reference)
import jax
import jax.numpy as jnp
from jax import lax
from jax.experimental import pallas as pl
from jax.experimental.pallas import tpu as pltpu


def _gather_exchange_add(safe_ids, mask, E):
    t = safe_ids.shape[0]
    d = E.shape[1]

    def body(ids_ref, mask_ref, e_ref, out_ref, partial_ref, comm_ref,
             gather_sem, send_sem, recv_sem):
        my_x = lax.axis_index("x")
        my_y = lax.axis_index("y")
        my_z = lax.axis_index("z")
        nbr = (my_x, 1 - my_y, my_z)

        def issue(i, _):
            pltpu.make_async_copy(
                e_ref.at[pl.ds(ids_ref[i], 1), :],
                partial_ref.at[pl.ds(i, 1), :],
                gather_sem,
            ).start()
            return _

        lax.fori_loop(0, t, issue, None)

        barrier_sem = pltpu.get_barrier_semaphore()
        pl.semaphore_signal(
            barrier_sem, inc=1, device_id=nbr,
            device_id_type=pl.DeviceIdType.MESH,
        )
        pl.semaphore_wait(barrier_sem, 1)

        def drain(i, _):
            pltpu.make_async_copy(
                e_ref.at[pl.ds(0, 1), :],
                partial_ref.at[pl.ds(0, 1), :],
                gather_sem,
            ).wait()
            return _

        lax.fori_loop(0, t, drain, None)

        partial_ref[...] = partial_ref[...] * mask_ref[...]

        rdma = pltpu.make_async_remote_copy(
            src_ref=partial_ref,
            dst_ref=comm_ref,
            send_sem=send_sem,
            recv_sem=recv_sem,
            device_id=nbr,
            device_id_type=pl.DeviceIdType.MESH,
        )
        rdma.start()
        rdma.wait()

        out_ref[...] = partial_ref[...] + comm_ref[...]

    return pl.pallas_call(
        body,
        out_shape=jax.ShapeDtypeStruct((t, d), jnp.float32),
        in_specs=[
            pl.BlockSpec(memory_space=pltpu.SMEM),
            pl.BlockSpec(memory_space=pltpu.VMEM),
            pl.BlockSpec(memory_space=pl.ANY),
        ],
        out_specs=pl.BlockSpec(memory_space=pltpu.VMEM),
        scratch_shapes=[
            pltpu.VMEM((t, d), jnp.float32),
            pltpu.VMEM((t, d), jnp.float32),
            pltpu.SemaphoreType.DMA,
            pltpu.SemaphoreType.DMA,
            pltpu.SemaphoreType.DMA,
        ],
        compiler_params=pltpu.CompilerParams(collective_id=0),
    )(safe_ids, mask, E)


def kernel(ids, E):
    my_y = lax.axis_index("y")
    v_shard = E.shape[0]
    local = ids - my_y * v_shard
    in_range = (local >= 0) & (local < v_shard)
    safe = jnp.where(in_range, local, 0).astype(jnp.int32)
    mask = in_range[:, None].astype(jnp.float32)
    return _gather_exchange_add(safe, mask, E)


# device time: 143297 ns/iter; 1.0932x vs baseline; 1.0932x over previous
import jax
import jax.numpy as jnp
from jax import lax
from jax.experimental import pallas as pl
from jax.experimental.pallas import tpu as pltpu

N_CHUNKS = 8


def _gather_exchange_select(safe_ids, mask, E):
    t = safe_ids.shape[0]
    d = E.shape[1]
    rows = t // N_CHUNKS

    def body(ids_ref, mask_ref, e_ref, out_ref, partial_ref, comm_ref,
             gather_sems, send_sems, recv_sems):
        my_x = lax.axis_index("x")
        my_y = lax.axis_index("y")
        my_z = lax.axis_index("z")
        nbr = (my_x, 1 - my_y, my_z)

        for c in range(N_CHUNKS):
            def issue(i, _, c=c):
                pltpu.make_async_copy(
                    e_ref.at[pl.ds(ids_ref[c * rows + i], 1), :],
                    partial_ref.at[pl.ds(c * rows + i, 1), :],
                    gather_sems.at[c],
                ).start()
                return _

            lax.fori_loop(0, rows, issue, None)

        barrier_sem = pltpu.get_barrier_semaphore()
        pl.semaphore_signal(
            barrier_sem, inc=1, device_id=nbr,
            device_id_type=pl.DeviceIdType.MESH,
        )
        pl.semaphore_wait(barrier_sem, 1)

        def chunk_rdma(c):
            return pltpu.make_async_remote_copy(
                src_ref=partial_ref.at[pl.ds(c * rows, rows), :],
                dst_ref=comm_ref.at[pl.ds(c * rows, rows), :],
                send_sem=send_sems.at[c],
                recv_sem=recv_sems.at[c],
                device_id=nbr,
                device_id_type=pl.DeviceIdType.MESH,
            )

        for c in range(N_CHUNKS):
            def drain(i, _, c=c):
                pltpu.make_async_copy(
                    e_ref.at[pl.ds(0, 1), :],
                    partial_ref.at[pl.ds(0, 1), :],
                    gather_sems.at[c],
                ).wait()
                return _

            lax.fori_loop(0, rows, drain, None)
            chunk_rdma(c).start()

        for c in range(N_CHUNKS):
            chunk_rdma(c).wait_recv()
            sl = (pl.ds(c * rows, rows), slice(None))
            out_ref[sl] = jnp.where(
                mask_ref[sl] > 0.5, partial_ref[sl], comm_ref[sl]
            )

        for c in range(N_CHUNKS):
            chunk_rdma(c).wait_send()

    return pl.pallas_call(
        body,
        out_shape=jax.ShapeDtypeStruct((t, d), jnp.float32),
        in_specs=[
            pl.BlockSpec(memory_space=pltpu.SMEM),
            pl.BlockSpec(memory_space=pltpu.VMEM),
            pl.BlockSpec(memory_space=pl.ANY),
        ],
        out_specs=pl.BlockSpec(memory_space=pltpu.VMEM),
        scratch_shapes=[
            pltpu.VMEM((t, d), jnp.float32),
            pltpu.VMEM((t, d), jnp.float32),
            pltpu.SemaphoreType.DMA((N_CHUNKS,)),
            pltpu.SemaphoreType.DMA((N_CHUNKS,)),
            pltpu.SemaphoreType.DMA((N_CHUNKS,)),
        ],
        compiler_params=pltpu.CompilerParams(collective_id=0),
    )(safe_ids, mask, E)


def kernel(ids, E):
    my_y = lax.axis_index("y")
    v_shard = E.shape[0]
    local = ids - my_y * v_shard
    in_range = (local >= 0) & (local < v_shard)
    safe = jnp.where(in_range, local, 0).astype(jnp.int32)
    mask = in_range[:, None].astype(jnp.float32)
    return _gather_exchange_select(safe, mask, E)


# device time: 106407 ns/iter; 1.4722x vs baseline; 1.3467x over previous
import jax
import jax.numpy as jnp
from jax import lax
from jax.experimental import pallas as pl
from jax.experimental.pallas import tpu as pltpu

N_CHUNKS = 8


def _gather_exchange_select(safe_ids, mask, E):
    t = safe_ids.shape[0]
    d = E.shape[1]
    rows = t // N_CHUNKS

    def body(ids_ref, mask_ref, e_ref, out_ref, partial_ref, comm_ref,
             gather_sems, send_sems, recv_sems):
        my_x = lax.axis_index("x")
        my_y = lax.axis_index("y")
        my_z = lax.axis_index("z")
        nbr = (my_x, 1 - my_y, my_z)

        def issue_chunk(c):
            def issue(i, _):
                pltpu.make_async_copy(
                    e_ref.at[pl.ds(ids_ref[c * rows + i], 1), :],
                    partial_ref.at[pl.ds(c * rows + i, 1), :],
                    gather_sems.at[c],
                ).start()
                return _

            lax.fori_loop(0, rows, issue, None)

        issue_chunk(0)

        barrier_sem = pltpu.get_barrier_semaphore()
        pl.semaphore_signal(
            barrier_sem, inc=1, device_id=nbr,
            device_id_type=pl.DeviceIdType.MESH,
        )
        pl.semaphore_wait(barrier_sem, 1)

        def chunk_rdma(c):
            return pltpu.make_async_remote_copy(
                src_ref=partial_ref.at[pl.ds(c * rows, rows), :],
                dst_ref=comm_ref.at[pl.ds(c * rows, rows), :],
                send_sem=send_sems.at[c],
                recv_sem=recv_sems.at[c],
                device_id=nbr,
                device_id_type=pl.DeviceIdType.MESH,
            )

        for c in range(N_CHUNKS):
            def drain(i, _, c=c):
                pltpu.make_async_copy(
                    e_ref.at[pl.ds(0, 1), :],
                    partial_ref.at[pl.ds(0, 1), :],
                    gather_sems.at[c],
                ).wait()
                return _

            lax.fori_loop(0, rows, drain, None)
            chunk_rdma(c).start()
            if c + 1 < N_CHUNKS:
                issue_chunk(c + 1)

        for c in range(N_CHUNKS):
            chunk_rdma(c).wait_recv()
            sl = (pl.ds(c * rows, rows), slice(None))
            out_ref[sl] = jnp.where(
                mask_ref[sl] > 0.5, partial_ref[sl], comm_ref[sl]
            )

        for c in range(N_CHUNKS):
            chunk_rdma(c).wait_send()

    return pl.pallas_call(
        body,
        out_shape=jax.ShapeDtypeStruct((t, d), jnp.float32),
        in_specs=[
            pl.BlockSpec(memory_space=pltpu.SMEM),
            pl.BlockSpec(memory_space=pltpu.VMEM),
            pl.BlockSpec(memory_space=pl.ANY),
        ],
        out_specs=pl.BlockSpec(memory_space=pltpu.VMEM),
        scratch_shapes=[
            pltpu.VMEM((t, d), jnp.float32),
            pltpu.VMEM((t, d), jnp.float32),
            pltpu.SemaphoreType.DMA((N_CHUNKS,)),
            pltpu.SemaphoreType.DMA((N_CHUNKS,)),
            pltpu.SemaphoreType.DMA((N_CHUNKS,)),
        ],
        compiler_params=pltpu.CompilerParams(collective_id=0),
    )(safe_ids, mask, E)


def kernel(ids, E):
    my_y = lax.axis_index("y")
    v_shard = E.shape[0]
    local = ids - my_y * v_shard
    in_range = (local >= 0) & (local < v_shard)
    safe = jnp.where(in_range, local, 0).astype(jnp.int32)
    mask = in_range[:, None].astype(jnp.float32)
    return _gather_exchange_select(safe, mask, E)
